# baseline (speedup 1.0000x reference)
import jax
import jax.numpy as jnp
from jax.experimental import pallas as pl
from jax.experimental.pallas import tpu as pltpu

_C_BLK = 64
_H = 128
_W = 128


def _scan_rows(v):
    c, m, n = v.shape
    neg_inf = jnp.float32(-jnp.inf)
    for s in (1, 2, 4):
        pad = jnp.full((c, s, n), neg_inf, jnp.float32)
        shifted = jnp.concatenate([pad, v[:, : m - s, :]], axis=1)
        v = jnp.maximum(v, shifted)
    acc = v[:, 0:8, :]
    parts = [acc]
    for i in range(1, m // 8):
        acc = jnp.maximum(v[:, 8 * i : 8 * (i + 1), :], acc)
        parts.append(acc)
    return jnp.concatenate(parts, axis=1)


def _pool_body(x_ref, o_ref):
    v = _scan_rows(x_ref[...])
    vt = jnp.swapaxes(v, 1, 2)
    vt = _scan_rows(vt)
    o_ref[...] = jnp.swapaxes(vt, 1, 2)


def kernel(x):
    b, c, h, w = x.shape
    n = b * c
    blk = min(_C_BLK, n)
    xr = x.reshape(n, h, w)
    out = pl.pallas_call(
        _pool_body,
        grid=(n // blk,),
        in_specs=[pl.BlockSpec((blk, h, w), lambda i: (i, 0, 0))],
        out_specs=pl.BlockSpec((blk, h, w), lambda i: (i, 0, 0)),
        out_shape=jax.ShapeDtypeStruct((n, h, w), x.dtype),
        compiler_params=pltpu.CompilerParams(
            dimension_semantics=("parallel",),
        ),
    )(xr)
    return out.reshape(b, c, h, w)


# speedup vs baseline: 1.0740x; 1.0740x over previous
import jax
import jax.numpy as jnp
from jax.experimental import pallas as pl
from jax.experimental.pallas import tpu as pltpu

_C_BLK = 128
_H = 128
_W = 128


def _scan_rows(v):
    c, m, n = v.shape
    neg_inf = jnp.float32(-jnp.inf)
    for s in (1, 2, 4):
        pad = jnp.full((c, s, n), neg_inf, jnp.float32)
        shifted = jnp.concatenate([pad, v[:, : m - s, :]], axis=1)
        v = jnp.maximum(v, shifted)
    acc = v[:, 0:8, :]
    parts = [acc]
    for i in range(1, m // 8):
        acc = jnp.maximum(v[:, 8 * i : 8 * (i + 1), :], acc)
        parts.append(acc)
    return jnp.concatenate(parts, axis=1)


def _pool_body(x_ref, o_ref):
    v = _scan_rows(x_ref[...])
    vt = jnp.swapaxes(v, 1, 2)
    vt = _scan_rows(vt)
    o_ref[...] = jnp.swapaxes(vt, 1, 2)


def kernel(x):
    b, c, h, w = x.shape
    n = b * c
    blk = min(_C_BLK, n)
    xr = x.reshape(n, h, w)
    out = pl.pallas_call(
        _pool_body,
        grid=(n // blk,),
        in_specs=[pl.BlockSpec((blk, h, w), lambda i: (i, 0, 0))],
        out_specs=pl.BlockSpec((blk, h, w), lambda i: (i, 0, 0)),
        out_shape=jax.ShapeDtypeStruct((n, h, w), x.dtype),
        compiler_params=pltpu.CompilerParams(
            dimension_semantics=("parallel",),
            vmem_limit_bytes=56 * 1024 * 1024,
        ),
    )(xr)
    return out.reshape(b, c, h, w)
